# baseline (speedup 1.0000x reference)
import functools

import jax
import jax.numpy as jnp
from jax import lax
from jax.experimental import pallas as pl
from jax.experimental.pallas import tpu as pltpu
from jax.experimental.pallas import tpu_sc as plsc

N = 256
R = N * N
NC = 2
NS = 16
NW = NC * NS
WR = R // NW
CR = 128
NCH = WR // CR
NBUF = 2


def _fix_chunk(buf, base_row):
    l = lax.iota(jnp.int32, 16)
    zero = l * 0
    one = zero + 1
    for j in range(CR // 16):
        rr = l + jnp.int32(base_row + 16 * j)
        v = plsc.load_gather(buf, [rr, one])
        plsc.store_scatter(buf, [rr, zero], v)


def _body(x_ref, o_ref, buf, in_sem, out_sem):
    wid = lax.axis_index("s") * NC + lax.axis_index("c")
    w0 = wid * WR

    def in_copy(i, s):
        return pltpu.make_async_copy(
            x_ref.at[pl.ds(w0 + i * CR, CR), :],
            buf.at[pl.ds(s * CR, CR), :],
            in_sem.at[s],
        )

    def out_copy(i, s):
        return pltpu.make_async_copy(
            buf.at[pl.ds(s * CR, CR), :],
            o_ref.at[pl.ds(w0 + i * CR, CR), :],
            out_sem.at[s],
        )

    in_copy(0, 0).start()
    for i in range(NCH):
        s = i % NBUF
        in_copy(i, s).wait()
        _fix_chunk(buf, s * CR)
        out_copy(i, s).start()
        nxt = i + 1
        if nxt < NCH:
            sn = nxt % NBUF
            if nxt >= NBUF:
                out_copy(nxt - NBUF, sn).wait()
            in_copy(nxt, sn).start()
    for k in range(max(0, NCH - NBUF), NCH):
        out_copy(k, k % NBUF).wait()


@functools.partial(jax.jit, static_argnums=())
def _sc_call(bs):
    mesh = plsc.VectorSubcoreMesh(core_axis_name="c", subcore_axis_name="s")
    f = functools.partial(
        pl.kernel,
        mesh=mesh,
        out_type=jax.ShapeDtypeStruct((R, N), jnp.float32),
        scratch_types=[
            pltpu.VMEM((NBUF * CR, N), jnp.float32),
            pltpu.SemaphoreType.DMA((NBUF,)),
            pltpu.SemaphoreType.DMA((NBUF,)),
        ],
        compiler_params=pltpu.CompilerParams(needs_layout_passes=False),
    )(_body)
    return f(bs)


def kernel(b):
    bs = b.reshape(R, N)
    out = _sc_call(bs)
    return out.reshape(1, N, N, N, 1)
